# baseline (speedup 1.0000x reference)
import functools

import jax
import jax.numpy as jnp
from jax import lax
from jax.experimental import pallas as pl
from jax.experimental.pallas import tpu as pltpu
from jax.experimental.pallas import tpu_sc as plsc

EMB = 64
HID = 64
NC, NS = 2, 16
NW = NC * NS
CHUNK = 128
PBLK = 4096


def _tc_project_table(table_t, W):
    V = table_t.shape[1]
    grid = (V + PBLK - 1) // PBLK

    def body(et_ref, w_ref, p_ref):
        y = lax.dot_general(et_ref[...], w_ref[...], (((0,), (1,)), ((), ())),
                            preferred_element_type=jnp.float32)
        p_ref[...] = jnp.concatenate(
            [y, jnp.zeros((PBLK, 128 - HID), jnp.float32)], axis=1)

    return pl.pallas_call(
        body,
        grid=(grid,),
        in_specs=[
            pl.BlockSpec((EMB, PBLK), lambda i: (0, i)),
            pl.BlockSpec((HID, EMB), lambda i: (0, 0)),
        ],
        out_specs=pl.BlockSpec((PBLK, 128), lambda i: (i, 0)),
        out_shape=jax.ShapeDtypeStruct((V, 128), jnp.float32),
    )(table_t, W)


def _sc_gather(idx3d, table):
    cpw = idx3d.shape[1]
    n_rows = NW * cpw * CHUNK

    mesh = plsc.VectorSubcoreMesh(core_axis_name="c", subcore_axis_name="s")

    @functools.partial(
        pl.kernel,
        out_type=jax.ShapeDtypeStruct((n_rows, 128), jnp.float32),
        mesh=mesh,
        scratch_types=[
            pltpu.VMEM((cpw, CHUNK), jnp.int32),
            pltpu.VMEM((2, CHUNK, 128), jnp.float32),
            pltpu.SemaphoreType.DMA,
            pltpu.SemaphoreType.DMA,
            pltpu.SemaphoreType.DMA,
            pltpu.SemaphoreType.DMA,
        ],
    )
    def k(idx_hbm, table_hbm, out_hbm, idx_v, rows_v, gs0, gs1, os0, os1):
        wid = lax.axis_index("s") * NC + lax.axis_index("c")
        gsems = (gs0, gs1)
        osems = (os0, os1)

        pltpu.sync_copy(idx_hbm.at[wid], idx_v)

        def gather(g, b):
            return pltpu.make_async_copy(
                table_hbm.at[idx_v.at[g]], rows_v.at[b], gsems[b])

        def scatter(g, b):
            return pltpu.make_async_copy(
                rows_v.at[b],
                out_hbm.at[pl.ds((wid * cpw + g) * CHUNK, CHUNK)],
                osems[b])

        gather(0, 0).start()

        @pl.loop(0, cpw, step=2)
        def _(i):
            for b in (0, 1):
                g = i + b
                @pl.when(g + 1 < cpw)
                def _():
                    @pl.when(g >= 1)
                    def _():
                        scatter(g - 1, 1 - b).wait()
                    gather(g + 1, 1 - b).start()

                gather(g, b).wait()
                scatter(g, b).start()

        scatter(cpw - 2, 0).wait()
        scatter(cpw - 1, 1).wait()

    return k(idx3d, table)


def _tc_finalize(g, batch, seq):
    def body(x_ref, o_ref):
        o_ref[...] = jnp.transpose(x_ref[...])[None]

    return pl.pallas_call(
        body,
        grid=(seq,),
        in_specs=[pl.BlockSpec((batch, HID), lambda i: (i, 0))],
        out_specs=pl.BlockSpec((1, HID, batch), lambda i: (i, 0, 0)),
        out_shape=jax.ShapeDtypeStruct((seq, HID, batch), jnp.float32),
    )(g)


def kernel(sent1, sent2, embedding, W):
    batch, seq = sent1.shape
    proj = _tc_project_table(embedding.T, W)
    idx1 = sent1.T.reshape(NW, -1, CHUNK).astype(jnp.int32)
    idx2 = sent2.T.reshape(NW, -1, CHUNK).astype(jnp.int32)
    g1 = _sc_gather(idx1, proj)
    g2 = _sc_gather(idx2, proj)
    t1 = _tc_finalize(g1[:, :HID], batch, seq)
    t2 = _tc_finalize(g2[:, :HID], batch, seq)
    return (jnp.transpose(t1, (2, 0, 1)), jnp.transpose(t2, (2, 0, 1)))
